# Initial kernel scaffold; baseline (speedup 1.0000x reference)
import jax
import jax.numpy as jnp
from jax.experimental import pallas as pl


def kernel(x, edge_index, edge_weight, W0):
    raise NotImplementedError("write your pallas kernel here")


# speedup vs baseline: 3.2834x; 3.2834x over previous
import functools

import jax
import jax.numpy as jnp
from jax import lax
from jax.experimental import pallas as pl
from jax.experimental.pallas import tpu as pltpu
from jax.experimental.pallas import tpu_sc as plsc

N_NODES = 10000
D = 128

NC = 2
NS = 16
NW = NC * NS
EC = 80
CHUNK = 128
EB = EC // 2
E_PAD = NW * EC * CHUNK

N_ACC = 10240
SUB_ROWS = N_ACC // NS


def _sc_edge_body(src_hbm, dst_hbm, w_hbm, x_hbm, out_hbm,
                  src_v, dst_v, w_v, rows, acc, sg0, sg1):
    sg = (sg0, sg1)
    c = lax.axis_index("c")
    s = lax.axis_index("s")
    wid = c * NS + s

    def zrow(r, carry):
        for k in range(D // 16):
            rows[0, r, pl.ds(k * 16, 16)] = jnp.zeros((16,), jnp.float32)
        return carry
    lax.fori_loop(0, CHUNK, zrow, 0)
    for t in range(SUB_ROWS // CHUNK):
        pltpu.sync_copy(rows.at[0],
                        acc.at[pl.ds(s * SUB_ROWS + t * CHUNK, CHUNK)])
    plsc.subcore_barrier()

    def _scale(j, b):
        def group(g, gcarry):
            w16 = w_v[j, pl.ds(g * 16, 16)]
            dnums = lax.GatherDimensionNumbers(
                offset_dims=(), collapsed_slice_dims=(0,),
                start_index_map=(0,))
            for t in range(16):
                wv = lax.gather(
                    w16, jnp.full((16, 1), t, jnp.int32), dnums,
                    slice_sizes=(1,),
                    mode=lax.GatherScatterMode.PROMISE_IN_BOUNDS)
                e = g * 16 + t
                for r in range(D // 16):
                    rows[b, e, pl.ds(r * 16, 16)] = (
                        rows[b, e, pl.ds(r * 16, 16)] * wv)
            return gcarry
        lax.fori_loop(0, CHUNK // 16, group, 0)

    for h in range(EC // EB):
        pltpu.sync_copy(src_hbm.at[wid, pl.ds(h * EB, EB)], src_v)
        pltpu.sync_copy(dst_hbm.at[wid, pl.ds(h * EB, EB)], dst_v)
        pltpu.sync_copy(w_hbm.at[wid, pl.ds(h * EB, EB)], w_v)

        pltpu.async_copy(x_hbm.at[src_v.at[0]], rows.at[0], sg[0])

        def pair(j2, carry):
            for b in range(2):
                j = j2 * 2 + b

                @pl.when(j + 1 < EB)
                def _start_next():
                    pltpu.async_copy(
                        x_hbm.at[src_v.at[j + 1]], rows.at[1 - b], sg[1 - b])

                pltpu.make_async_copy(
                    x_hbm.at[src_v.at[j]], rows.at[b], sg[b]).wait()
                _scale(j, b)
                pltpu.sync_copy(rows.at[b], acc.at[dst_v.at[j]], add=True)
            return carry
        lax.fori_loop(0, EB // 2, pair, 0)

    plsc.subcore_barrier()

    pltpu.sync_copy(acc.at[pl.ds(s * SUB_ROWS, SUB_ROWS)],
                    out_hbm.at[c, pl.ds(s * SUB_ROWS, SUB_ROWS)])


_sc_edge = functools.partial(
    pl.kernel,
    out_type=jax.ShapeDtypeStruct((NC, N_ACC, D), jnp.float32),
    mesh=plsc.VectorSubcoreMesh(core_axis_name="c", subcore_axis_name="s"),
    scratch_types=[
        pltpu.VMEM((EB, CHUNK), jnp.int32),
        pltpu.VMEM((EB, CHUNK), jnp.int32),
        pltpu.VMEM((EB, CHUNK), jnp.float32),
        pltpu.VMEM((2, CHUNK, D), jnp.float32),
        pltpu.VMEM_SHARED((N_ACC, D), jnp.float32),
        pltpu.SemaphoreType.DMA,
        pltpu.SemaphoreType.DMA,
    ],
)(_sc_edge_body)


def _mm_body(p_ref, w_ref, o_ref):
    agg = p_ref[0] + p_ref[1]
    o_ref[...] = jnp.dot(agg, w_ref[...], preferred_element_type=jnp.float32)


_MM_BLOCK = 1000


def _mm(parts, W0):
    grid = (N_NODES // _MM_BLOCK,)
    return pl.pallas_call(
        _mm_body,
        grid=grid,
        in_specs=[
            pl.BlockSpec((NC, _MM_BLOCK, D), lambda i: (0, i, 0)),
            pl.BlockSpec((D, D), lambda i: (0, 0)),
        ],
        out_specs=pl.BlockSpec((_MM_BLOCK, D), lambda i: (i, 0)),
        out_shape=jax.ShapeDtypeStruct((N_NODES, D), jnp.float32),
    )(parts, W0)


@jax.jit
def kernel(x, edge_index, edge_weight, W0):
    src = edge_index[0].astype(jnp.int32)
    dst = edge_index[1].astype(jnp.int32)
    ew = edge_weight.astype(jnp.float32)
    e = src.shape[0]
    pad = E_PAD - e
    src = jnp.concatenate([src, jnp.zeros((pad,), jnp.int32)])
    dst = jnp.concatenate([dst, jnp.zeros((pad,), jnp.int32)])
    ew = jnp.concatenate([ew, jnp.zeros((pad,), jnp.float32)])
    src = src.reshape(NW, EC, CHUNK)
    dst = dst.reshape(NW, EC, CHUNK)
    ew = ew.reshape(NW, EC, CHUNK)
    parts = _sc_edge(src, dst, ew, x)
    return _mm(parts, W0)


# speedup vs baseline: 3.5263x; 1.0740x over previous
import functools

import jax
import jax.numpy as jnp
from jax import lax
from jax.experimental import pallas as pl
from jax.experimental.pallas import tpu as pltpu
from jax.experimental.pallas import tpu_sc as plsc

N_NODES = 10000
D = 128

NC = 2
NS = 16
NW = NC * NS
CHUNK = 128

K0 = 128
K1 = 32
EB = 32
NCHUNKS = NS * (K0 + K1)
CH0 = NS * K0
E_PAD = NCHUNKS * CHUNK

N_ACC = 10240
SUB_ROWS = N_ACC // NS


def _sc_edge_body(src_hbm, dst_hbm, w_hbm, x_hbm, out_hbm,
                  src_v, dst_v, w_v, rows, acc, sg0, sg1):
    sg = (sg0, sg1)
    c = lax.axis_index("c")
    s = lax.axis_index("s")

    def zrow(r, carry):
        for k in range(D // 16):
            rows[0, r, pl.ds(k * 16, 16)] = jnp.zeros((16,), jnp.float32)
        return carry
    lax.fori_loop(0, CHUNK, zrow, 0)
    for t in range(SUB_ROWS // CHUNK):
        pltpu.sync_copy(rows.at[0],
                        acc.at[pl.ds(s * SUB_ROWS + t * CHUNK, CHUNK)])
    plsc.subcore_barrier()

    def _scale(j, b):
        def group(g, gcarry):
            w16 = w_v[j, pl.ds(g * 16, 16)]
            dnums = lax.GatherDimensionNumbers(
                offset_dims=(), collapsed_slice_dims=(0,),
                start_index_map=(0,))
            for t in range(16):
                wv = lax.gather(
                    w16, jnp.full((16, 1), t, jnp.int32), dnums,
                    slice_sizes=(1,),
                    mode=lax.GatherScatterMode.PROMISE_IN_BOUNDS)
                e = g * 16 + t
                for r in range(D // 16):
                    rows[b, e, pl.ds(r * 16, 16)] = (
                        rows[b, e, pl.ds(r * 16, 16)] * wv)
            return gcarry
        lax.fori_loop(0, CHUNK // 16, group, 0)

    base = jnp.where(c == 0, s * K0, CH0 + s * K1)
    nblk = jnp.where(c == 0, K0 // EB, K1 // EB)

    def block(h, carry):
        blk = base + h * EB
        pltpu.sync_copy(src_hbm.at[pl.ds(blk, EB)], src_v)
        pltpu.sync_copy(dst_hbm.at[pl.ds(blk, EB)], dst_v)
        pltpu.sync_copy(w_hbm.at[pl.ds(blk, EB)], w_v)

        pltpu.async_copy(x_hbm.at[src_v.at[0]], rows.at[0], sg[0])

        def pair(j2, icarry):
            for b in range(2):
                j = j2 * 2 + b

                @pl.when(j + 1 < EB)
                def _start_next():
                    pltpu.async_copy(
                        x_hbm.at[src_v.at[j + 1]], rows.at[1 - b], sg[1 - b])

                pltpu.make_async_copy(
                    x_hbm.at[src_v.at[j]], rows.at[b], sg[b]).wait()
                _scale(j, b)
                pltpu.sync_copy(rows.at[b], acc.at[dst_v.at[j]], add=True)
            return icarry
        lax.fori_loop(0, EB // 2, pair, 0)
        return carry
    lax.fori_loop(0, nblk, block, 0)

    plsc.subcore_barrier()

    pltpu.sync_copy(acc.at[pl.ds(s * SUB_ROWS, SUB_ROWS)],
                    out_hbm.at[c, pl.ds(s * SUB_ROWS, SUB_ROWS)])


_sc_edge = functools.partial(
    pl.kernel,
    out_type=jax.ShapeDtypeStruct((NC, N_ACC, D), jnp.float32),
    mesh=plsc.VectorSubcoreMesh(core_axis_name="c", subcore_axis_name="s"),
    scratch_types=[
        pltpu.VMEM((EB, CHUNK), jnp.int32),
        pltpu.VMEM((EB, CHUNK), jnp.int32),
        pltpu.VMEM((EB, CHUNK), jnp.float32),
        pltpu.VMEM((2, CHUNK, D), jnp.float32),
        pltpu.VMEM_SHARED((N_ACC, D), jnp.float32),
        pltpu.SemaphoreType.DMA,
        pltpu.SemaphoreType.DMA,
    ],
)(_sc_edge_body)


def _mm_body(p_ref, w_ref, o_ref):
    agg = p_ref[0] + p_ref[1]
    o_ref[...] = jnp.dot(agg, w_ref[...], preferred_element_type=jnp.float32)


_MM_BLOCK = 1000


def _mm(parts, W0):
    grid = (N_NODES // _MM_BLOCK,)
    return pl.pallas_call(
        _mm_body,
        grid=grid,
        in_specs=[
            pl.BlockSpec((NC, _MM_BLOCK, D), lambda i: (0, i, 0)),
            pl.BlockSpec((D, D), lambda i: (0, 0)),
        ],
        out_specs=pl.BlockSpec((_MM_BLOCK, D), lambda i: (i, 0)),
        out_shape=jax.ShapeDtypeStruct((N_NODES, D), jnp.float32),
    )(parts, W0)


@jax.jit
def kernel(x, edge_index, edge_weight, W0):
    src = edge_index[0].astype(jnp.int32)
    dst = edge_index[1].astype(jnp.int32)
    ew = edge_weight.astype(jnp.float32)
    e = src.shape[0]
    pad = E_PAD - e
    src = jnp.concatenate([src, jnp.zeros((pad,), jnp.int32)])
    dst = jnp.concatenate([dst, jnp.zeros((pad,), jnp.int32)])
    ew = jnp.concatenate([ew, jnp.zeros((pad,), jnp.float32)])
    src = src.reshape(NCHUNKS, CHUNK)
    dst = dst.reshape(NCHUNKS, CHUNK)
    ew = ew.reshape(NCHUNKS, CHUNK)
    parts = _sc_edge(src, dst, ew, x)
    return _mm(parts, W0)


# speedup vs baseline: 10.6569x; 3.0221x over previous
import functools

import jax
import jax.numpy as jnp
from jax import lax
from jax.experimental import pallas as pl
from jax.experimental.pallas import tpu as pltpu
from jax.experimental.pallas import tpu_sc as plsc

N_NODES = 10000
D = 128

NC = 2
NS = 16
NW = NC * NS
CHUNK = 128

K0 = 80
K1 = 80
EB = 40
NCHUNKS = NS * (K0 + K1)
CH0 = NS * K0
E_PAD = NCHUNKS * CHUNK

N_ACC = 10240
SUB_ROWS = N_ACC // NS


def _sc_edge_body(src_hbm, dst_hbm, w_hbm, x_hbm, out_hbm,
                  src_v, dst_v, w_v, rows, acc, sg0, sg1, sh0, sh1):
    sg = (sg0, sg1)
    sh = (sh0, sh1)
    HC = CHUNK // 2
    c = lax.axis_index("c")
    s = lax.axis_index("s")

    def zrow(r, carry):
        for k in range(D // 16):
            rows[0, r, pl.ds(k * 16, 16)] = jnp.zeros((16,), jnp.float32)
        return carry
    lax.fori_loop(0, CHUNK, zrow, 0)
    for t in range(SUB_ROWS // CHUNK):
        pltpu.sync_copy(rows.at[0],
                        acc.at[pl.ds(s * SUB_ROWS + t * CHUNK, CHUNK)])
    plsc.subcore_barrier()

    def _scale(j, b):
        def group(g, gcarry):
            w16 = w_v[j, pl.ds(g * 16, 16)]
            dnums = lax.GatherDimensionNumbers(
                offset_dims=(), collapsed_slice_dims=(0,),
                start_index_map=(0,))
            for t in range(16):
                wv = lax.gather(
                    w16, jnp.full((16, 1), t, jnp.int32), dnums,
                    slice_sizes=(1,),
                    mode=lax.GatherScatterMode.PROMISE_IN_BOUNDS)
                e = g * 16 + t
                for r in range(D // 16):
                    rows[b, e, pl.ds(r * 16, 16)] = (
                        rows[b, e, pl.ds(r * 16, 16)] * wv)
            return gcarry
        lax.fori_loop(0, CHUNK // 16, group, 0)

    base = jnp.where(c == 0, s * K0, CH0 + s * K1)
    nblk = jnp.where(c == 0, K0 // EB, K1 // EB)

    def block(h, carry):
        blk = base + h * EB
        pltpu.sync_copy(src_hbm.at[pl.ds(blk, EB)], src_v)
        pltpu.sync_copy(dst_hbm.at[pl.ds(blk, EB)], dst_v)
        pltpu.sync_copy(w_hbm.at[pl.ds(blk, EB)], w_v)

        def _start_gather(j, b):
            pltpu.async_copy(
                x_hbm.at[src_v.at[j, pl.ds(0, HC)]],
                rows.at[b, pl.ds(0, HC)], sg[b])
            pltpu.async_copy(
                x_hbm.at[src_v.at[j, pl.ds(HC, HC)]],
                rows.at[b, pl.ds(HC, HC)], sh[b])

        def _wait_gather(j, b):
            pltpu.make_async_copy(
                x_hbm.at[src_v.at[j, pl.ds(0, HC)]],
                rows.at[b, pl.ds(0, HC)], sg[b]).wait()
            pltpu.make_async_copy(
                x_hbm.at[src_v.at[j, pl.ds(HC, HC)]],
                rows.at[b, pl.ds(HC, HC)], sh[b]).wait()

        _start_gather(0, 0)

        def pair(j2, icarry):
            for b in range(2):
                j = j2 * 2 + b

                @pl.when(j + 1 < EB)
                def _start_next():
                    _start_gather(j + 1, 1 - b)

                _wait_gather(j, b)
                _scale(j, b)
                pltpu.sync_copy(rows.at[b], acc.at[dst_v.at[j]], add=True)
            return icarry
        lax.fori_loop(0, EB // 2, pair, 0)
        return carry
    lax.fori_loop(0, nblk, block, 0)

    plsc.subcore_barrier()

    pltpu.sync_copy(acc.at[pl.ds(s * SUB_ROWS, SUB_ROWS)],
                    out_hbm.at[c, pl.ds(s * SUB_ROWS, SUB_ROWS)])


_sc_edge = functools.partial(
    pl.kernel,
    out_type=jax.ShapeDtypeStruct((NC, N_ACC, D), jnp.float32),
    mesh=plsc.VectorSubcoreMesh(core_axis_name="c", subcore_axis_name="s"),
    scratch_types=[
        pltpu.VMEM((EB, CHUNK), jnp.int32),
        pltpu.VMEM((EB, CHUNK), jnp.int32),
        pltpu.VMEM((EB, CHUNK), jnp.float32),
        pltpu.VMEM((2, CHUNK, D), jnp.float32),
        pltpu.VMEM_SHARED((N_ACC, D), jnp.float32),
        pltpu.SemaphoreType.DMA,
        pltpu.SemaphoreType.DMA,
        pltpu.SemaphoreType.DMA,
        pltpu.SemaphoreType.DMA,
    ],
)(_sc_edge_body)


def _mm_body(p_ref, w_ref, o_ref):
    agg = p_ref[0] + p_ref[1]
    o_ref[...] = jnp.dot(agg, w_ref[...], preferred_element_type=jnp.float32)


_MM_BLOCK = 1000


def _mm(parts, W0):
    grid = (N_NODES // _MM_BLOCK,)
    return pl.pallas_call(
        _mm_body,
        grid=grid,
        in_specs=[
            pl.BlockSpec((NC, _MM_BLOCK, D), lambda i: (0, i, 0)),
            pl.BlockSpec((D, D), lambda i: (0, 0)),
        ],
        out_specs=pl.BlockSpec((_MM_BLOCK, D), lambda i: (i, 0)),
        out_shape=jax.ShapeDtypeStruct((N_NODES, D), jnp.float32),
    )(parts, W0)


@jax.jit
def kernel(x, edge_index, edge_weight, W0):
    src = edge_index[0].astype(jnp.int32)
    dst = edge_index[1].astype(jnp.int32)
    ew = edge_weight.astype(jnp.float32)
    e = src.shape[0]
    pad = E_PAD - e
    spread = jnp.arange(pad, dtype=jnp.int32)
    src = jnp.concatenate([src, spread % N_NODES])
    dst = jnp.concatenate([dst, spread % N_ACC])
    ew = jnp.concatenate([ew, jnp.zeros((pad,), jnp.float32)])
    src = src.reshape(NCHUNKS, CHUNK)
    dst = dst.reshape(NCHUNKS, CHUNK)
    ew = ew.reshape(NCHUNKS, CHUNK)
    parts = _sc_edge(src, dst, ew, x)
    return _mm(parts, W0)
